# baseline (speedup 1.0000x reference)
import functools
import numpy as np
import jax
import jax.numpy as jnp
from jax import lax
from jax.experimental import pallas as pl
from jax.experimental.pallas import tpu as pltpu
from jax.experimental.pallas import tpu_sc as plsc

LV = 16
FP = 2
LOG2T = 19
TT = 1 << LOG2T
_BASE = 2 ** 4
_MAXR = 2 ** 11
_SCALE = np.exp((np.log(_MAXR) - np.log(_BASE)) / (LV - 1))
RESL = [int(np.floor(_BASE * _SCALE ** l)) for l in range(LV)]
PR1 = np.uint32(2654435761)
PR2 = np.uint32(805459861)

NC = 2
NS = 16
NW = NC * NS
L = 16

CH = 512
NG = CH // L
NI = 8 * CH
CORNERS = [(dx, dy, dz) for dx in (0, 1) for dy in (0, 1) for dz in (0, 1)]

_i32 = jnp.int32
_f32 = jnp.float32


def _iota():
    return lax.iota(_i32, L)


def _hash_make(n_pts):
    ptsw = n_pts // NW
    nch = ptsw // CH

    mesh = plsc.VectorSubcoreMesh(
        core_axis_name="c", subcore_axis_name="s", num_cores=NC,
        num_subcores=NS)

    @functools.partial(
        pl.kernel,
        out_type=jax.ShapeDtypeStruct((n_pts * LV * FP,), _f32),
        mesh=mesh,
        compiler_params=pltpu.CompilerParams(
            needs_layout_passes=False, use_tc_tiling_on_sc=False),
        scratch_types=[
            pltpu.VMEM((CH, 3), _f32),
            pltpu.VMEM((3 * CH,), _f32),
            pltpu.VMEM((3 * CH,), _f32),
            pltpu.VMEM((3 * CH,), _f32),
            pltpu.VMEM((NI,), _i32),
            pltpu.VMEM((NI,), _i32),
            pltpu.VMEM((NI, FP), _f32),
            pltpu.VMEM((NI, FP), _f32),
            pltpu.VMEM((CH * LV * FP,), _f32),
            pltpu.VMEM((LV * FP * L,), _f32),
            pltpu.SemaphoreType.DMA,
            pltpu.SemaphoreType.DMA,
        ],
    )
    def hashgrid(xt_hbm, tab_hbm, maskb_hbm, out_hbm, xv, xp, wv0, wv1,
                 iv0, iv1, rv0, rv1, ov, mv, sem0, sem1):
        wid = lax.axis_index("s") * NC + lax.axis_index("c")
        sems = (sem0, sem1)
        wvs = (wv0, wv1)
        ivs = (iv0, iv1)
        rvs = (rv0, rv1)
        pltpu.sync_copy(maskb_hbm, mv)

        def pass_a(lv, b):
            res = float(RESL[lv])
            lvoff = lv * TT
            wv = wvs[b]
            iv = ivs[b]

            def body(g, carry):
                o16 = g * L
                px = xp[pl.ds(o16, L)] * res
                py = xp[pl.ds(CH + o16, L)] * res
                pz = xp[pl.ds(2 * CH + o16, L)] * res
                ix = px.astype(_i32)
                iy = py.astype(_i32)
                iz = pz.astype(_i32)
                wv[pl.ds(o16, L)] = px - ix.astype(_f32)
                wv[pl.ds(CH + o16, L)] = py - iy.astype(_f32)
                wv[pl.ds(2 * CH + o16, L)] = pz - iz.astype(_f32)
                hx = (ix.astype(jnp.uint32), (ix + 1).astype(jnp.uint32))
                hy = (iy.astype(jnp.uint32) * PR1,
                      (iy + 1).astype(jnp.uint32) * PR1)
                hz = (iz.astype(jnp.uint32) * PR2,
                      (iz + 1).astype(jnp.uint32) * PR2)
                for c, (dx, dy, dz) in enumerate(CORNERS):
                    h = (hx[dx] ^ hy[dy] ^ hz[dz]) & jnp.uint32(TT - 1)
                    iv[pl.ds(c * CH + o16, L)] = h.astype(_i32) + lvoff
                return carry

            lax.fori_loop(0, NG, body, 0)

        def start_stream(b):
            return pltpu.async_copy(tab_hbm.at[ivs[b]], rvs[b], sems[b])

        def pass_b(lv, b):
            wv = wvs[b]
            rv = rvs[b]

            def body(g, carry):
                o16 = g * L
                fx = wv[pl.ds(o16, L)]
                fy = wv[pl.ds(CH + o16, L)]
                fz = wv[pl.ds(2 * CH + o16, L)]
                gx = (1.0 - fx, fx)
                gy = (1.0 - fy, fy)
                gz = (1.0 - fz, fz)
                wxy = [[gx[0] * gy[0], gx[0] * gy[1]],
                       [gx[1] * gy[0], gx[1] * gy[1]]]
                acc0 = jnp.zeros((L,), _f32)
                acc1 = jnp.zeros((L,), _f32)
                zc = jnp.zeros((L,), _i32)
                oc = zc + 1
                it = _iota()
                for c, (dx, dy, dz) in enumerate(CORNERS):
                    rid = it + (c * CH + o16)
                    f0 = plsc.load_gather(rv, [rid, zc])
                    f1 = plsc.load_gather(rv, [rid, oc])
                    wc = wxy[dx][dy] * gz[dz]
                    acc0 = acc0 + wc * f0
                    acc1 = acc1 + wc * f1
                acc0 = acc0 * mv[pl.ds(2 * lv * L, L)]
                acc1 = acc1 * mv[pl.ds((2 * lv + 1) * L, L)]
                oidx = (it << 5) + (o16 * LV * FP + 2 * lv)
                plsc.store_scatter(ov, [oidx], acc0)
                plsc.store_scatter(ov, [oidx + 1], acc1)
                return carry

            lax.fori_loop(0, NG, body, 0)

        def chunk(ci, carry):
            base = wid * ptsw + ci * CH
            pltpu.sync_copy(xt_hbm.at[pl.ds(base, CH)], xv)

            def tbody(g, carry):
                o16 = g * L
                itp = _iota() + o16
                zc3 = jnp.zeros((L,), _i32)
                xp[pl.ds(o16, L)] = plsc.load_gather(xv, [itp, zc3])
                xp[pl.ds(CH + o16, L)] = plsc.load_gather(xv, [itp, zc3 + 1])
                xp[pl.ds(2 * CH + o16, L)] = plsc.load_gather(
                    xv, [itp, zc3 + 2])
                return carry

            lax.fori_loop(0, NG, tbody, 0)
            pass_a(0, 0)
            handles = {0: start_stream(0), 1: None}
            for lv in range(1, LV):
                b = lv & 1
                pass_a(lv, b)
                handles[b] = start_stream(b)
                handles[1 - b].wait()
                pass_b(lv - 1, 1 - b)
            handles[1].wait()
            pass_b(LV - 1, 1)
            pltpu.sync_copy(ov, out_hbm.at[pl.ds(base * LV * FP, CH * LV * FP)])
            return carry

        lax.fori_loop(0, nch, chunk, 0)

    return hashgrid


def kernel(x, table, mask):
    n_pts = x.shape[0]
    xt = x
    tab = table.reshape(LV * TT, FP)
    maskb = jnp.broadcast_to(mask.astype(_f32).reshape(LV * FP, 1),
                             (LV * FP, L)).reshape(-1)
    out = _hash_make(n_pts)(xt, tab, maskb)
    return out.reshape(n_pts, LV * FP)


# speedup vs baseline: 3.2351x; 3.2351x over previous
import functools
import numpy as np
import jax
import jax.numpy as jnp
from jax import lax
from jax.experimental import pallas as pl
from jax.experimental.pallas import tpu as pltpu
from jax.experimental.pallas import tpu_sc as plsc

LV = 16
FP = 2
LOG2T = 19
TT = 1 << LOG2T
_BASE = 2 ** 4
_MAXR = 2 ** 11
_SCALE = np.exp((np.log(_MAXR) - np.log(_BASE)) / (LV - 1))
RESL = [int(np.floor(_BASE * _SCALE ** l)) for l in range(LV)]
PR1 = np.uint32(2654435761)
PR2 = np.uint32(805459861)

NC = 2
NS = 16
NW = NC * NS
L = 16

CH = 512
NG = CH // L
NI = 2 * 8 * CH
CORNERS = [(dx, dy, dz) for dx in (0, 1) for dy in (0, 1) for dz in (0, 1)]

_i32 = jnp.int32
_f32 = jnp.float32


def _hash_make(n_pts):
    ptsw = n_pts // NW
    nch = ptsw // CH

    mesh = plsc.VectorSubcoreMesh(
        core_axis_name="c", subcore_axis_name="s", num_cores=NC,
        num_subcores=NS)

    @functools.partial(
        pl.kernel,
        out_type=jax.ShapeDtypeStruct((LV * FP, n_pts), _f32),
        mesh=mesh,
        compiler_params=pltpu.CompilerParams(
            needs_layout_passes=False, use_tc_tiling_on_sc=False),
        scratch_types=[
            pltpu.VMEM((3 * CH,), _f32),
            pltpu.VMEM((3 * CH,), _f32),
            pltpu.VMEM((3 * CH,), _f32),
            pltpu.VMEM((NI,), _i32),
            pltpu.VMEM((NI,), _i32),
            pltpu.VMEM((NI,), _f32),
            pltpu.VMEM((NI,), _f32),
            pltpu.VMEM((LV * FP, CH), _f32),
            pltpu.VMEM((LV * FP * L,), _f32),
            pltpu.SemaphoreType.DMA,
            pltpu.SemaphoreType.DMA,
        ],
    )
    def hashgrid(xt_hbm, tab_hbm, maskb_hbm, out_hbm, xv, wv0, wv1,
                 iv0, iv1, rv0, rv1, ov, mv, sem0, sem1):
        wid = lax.axis_index("s") * NC + lax.axis_index("c")
        sems = (sem0, sem1)
        wvs = (wv0, wv1)
        ivs = (iv0, iv1)
        rvs = (rv0, rv1)
        pltpu.sync_copy(maskb_hbm, mv)

        def pass_a(lv, b):
            res = float(RESL[lv])
            p0off = (2 * lv) * TT
            wv = wvs[b]
            iv = ivs[b]

            def body(g, carry):
                o16 = g * L
                px = xv[pl.ds(o16, L)] * res
                py = xv[pl.ds(CH + o16, L)] * res
                pz = xv[pl.ds(2 * CH + o16, L)] * res
                ix = px.astype(_i32)
                iy = py.astype(_i32)
                iz = pz.astype(_i32)
                wv[pl.ds(o16, L)] = px - ix.astype(_f32)
                wv[pl.ds(CH + o16, L)] = py - iy.astype(_f32)
                wv[pl.ds(2 * CH + o16, L)] = pz - iz.astype(_f32)
                hx = (ix.astype(jnp.uint32), (ix + 1).astype(jnp.uint32))
                hy = (iy.astype(jnp.uint32) * PR1,
                      (iy + 1).astype(jnp.uint32) * PR1)
                hz = (iz.astype(jnp.uint32) * PR2,
                      (iz + 1).astype(jnp.uint32) * PR2)
                for c, (dx, dy, dz) in enumerate(CORNERS):
                    h = (hx[dx] ^ hy[dy] ^ hz[dz]) & jnp.uint32(TT - 1)
                    hi = h.astype(_i32)
                    e0 = (((hi & 0x7FF80) << 1) | (hi & 0x7F)) + p0off
                    iv[pl.ds(c * CH + o16, L)] = e0
                    iv[pl.ds(8 * CH + c * CH + o16, L)] = e0 + 128
                return carry

            lax.fori_loop(0, NG, body, 0)

        def start_stream(b):
            return pltpu.async_copy(tab_hbm.at[ivs[b]], rvs[b], sems[b])

        def pass_b(lv, b):
            wv = wvs[b]
            rv = rvs[b]

            def body(g, carry):
                o16 = g * L
                fx = wv[pl.ds(o16, L)]
                fy = wv[pl.ds(CH + o16, L)]
                fz = wv[pl.ds(2 * CH + o16, L)]
                gx = (1.0 - fx, fx)
                gy = (1.0 - fy, fy)
                gz = (1.0 - fz, fz)
                wxy = [[gx[0] * gy[0], gx[0] * gy[1]],
                       [gx[1] * gy[0], gx[1] * gy[1]]]
                acc0 = jnp.zeros((L,), _f32)
                acc1 = jnp.zeros((L,), _f32)
                for c, (dx, dy, dz) in enumerate(CORNERS):
                    f0 = rv[pl.ds(c * CH + o16, L)]
                    f1 = rv[pl.ds(8 * CH + c * CH + o16, L)]
                    wc = wxy[dx][dy] * gz[dz]
                    acc0 = acc0 + wc * f0
                    acc1 = acc1 + wc * f1
                acc0 = acc0 * mv[pl.ds(2 * lv * L, L)]
                acc1 = acc1 * mv[pl.ds((2 * lv + 1) * L, L)]
                ov[2 * lv, pl.ds(o16, L)] = acc0
                ov[2 * lv + 1, pl.ds(o16, L)] = acc1
                return carry

            lax.fori_loop(0, NG, body, 0)

        def chunk(ci, carry):
            base = wid * ptsw + ci * CH
            pltpu.sync_copy(xt_hbm.at[pl.ds(base, CH)], xv.at[pl.ds(0, CH)])
            pltpu.sync_copy(xt_hbm.at[pl.ds(n_pts + base, CH)],
                            xv.at[pl.ds(CH, CH)])
            pltpu.sync_copy(xt_hbm.at[pl.ds(2 * n_pts + base, CH)],
                            xv.at[pl.ds(2 * CH, CH)])
            pass_a(0, 0)
            handles = {0: start_stream(0), 1: None}
            for lv in range(1, LV):
                b = lv & 1
                pass_a(lv, b)
                handles[b] = start_stream(b)
                handles[1 - b].wait()
                pass_b(lv - 1, 1 - b)
            handles[1].wait()
            pass_b(LV - 1, 1)
            pltpu.sync_copy(ov, out_hbm.at[:, pl.ds(base, CH)])
            return carry

        lax.fori_loop(0, nch, chunk, 0)

    return hashgrid


def kernel(x, table, mask):
    n_pts = x.shape[0]
    xt = x.T.reshape(-1)
    tabp = table.reshape(LV, TT // 128, 128, FP).transpose(
        0, 1, 3, 2).reshape(-1)
    maskb = jnp.broadcast_to(mask.astype(_f32).reshape(LV * FP, 1),
                             (LV * FP, L)).reshape(-1)
    out = _hash_make(n_pts)(xt, tabp, maskb)
    return out.T
